# Initial kernel scaffold; baseline (speedup 1.0000x reference)
import functools

import jax
import jax.numpy as jnp
from jax import lax
from jax.experimental import pallas as pl
from jax.experimental.pallas import tpu as pltpu
from jax.experimental.pallas import tpu_sc as plsc

N_NODES = 10000
N_EDGES = 320000
D = 128
LANES = 16

NUM_CORES = 2
NUM_SUBCORES = 16
NUM_WORKERS = NUM_CORES * NUM_SUBCORES
EDGES_PER_WORKER = N_EDGES // NUM_WORKERS
CHUNK = 80
NUM_CHUNKS = EDGES_PER_WORKER // CHUNK
ROWS_PER_TILE = N_NODES // NUM_SUBCORES
ZROWS = 125

_SDS = jax.ShapeDtypeStruct


def _make_sc_agg(with_deg):
  mesh = plsc.VectorSubcoreMesh(
      core_axis_name="c", subcore_axis_name="s",
      num_cores=NUM_CORES, num_subcores=NUM_SUBCORES)

  out_type = [_SDS((NUM_CORES, N_NODES, D), jnp.float32)]
  scratch = [
      pltpu.VMEM_SHARED((N_NODES, D), jnp.float32),
      pltpu.VMEM((CHUNK,), jnp.int32),
      pltpu.VMEM((CHUNK,), jnp.int32),
      pltpu.VMEM((CHUNK, D), jnp.float32),
      pltpu.VMEM((ZROWS, D), jnp.float32),
      pltpu.SemaphoreType.DMA,
  ]
  if with_deg:
    out_type.append(_SDS((NUM_CORES, N_NODES, LANES), jnp.float32))
    scratch += [
        pltpu.VMEM_SHARED((N_NODES, LANES), jnp.float32),
        pltpu.VMEM((CHUNK, LANES), jnp.float32),
        pltpu.VMEM((ZROWS, LANES), jnp.float32),
    ]

  @functools.partial(pl.kernel, mesh=mesh, out_type=tuple(out_type),
                     scratch_types=tuple(scratch))
  def agg(x_hbm, src_hbm, dst_hbm, part_hbm, *rest):
    if with_deg:
      deg_hbm, acc, sidx, didx, rows, zbuf, sem, degacc, ones, zdeg = rest
    else:
      acc, sidx, didx, rows, zbuf, sem = rest

    c = lax.axis_index("c")
    s = lax.axis_index("s")
    wid = s * NUM_CORES + c

    zero16 = jnp.zeros((LANES,), jnp.float32)

    def zfill(k, carry):
      zbuf[k // (D // LANES), pl.ds((k % (D // LANES)) * LANES, LANES)] = zero16
      return carry
    lax.fori_loop(0, ZROWS * (D // LANES), zfill, 0)

    if with_deg:
      one16 = jnp.ones((LANES,), jnp.float32)

      def ofill(k, carry):
        ones[k] = one16
        return carry
      lax.fori_loop(0, CHUNK, ofill, 0)

      def zdfill(k, carry):
        zdeg[k] = zero16
        return carry
      lax.fori_loop(0, ZROWS, zdfill, 0)

    for r in range(ROWS_PER_TILE // ZROWS):
      pltpu.sync_copy(zbuf, acc.at[pl.ds(s * ROWS_PER_TILE + r * ZROWS, ZROWS)])
      if with_deg:
        pltpu.sync_copy(zdeg,
                        degacc.at[pl.ds(s * ROWS_PER_TILE + r * ZROWS, ZROWS)])
    plsc.subcore_barrier()

    base_w = wid * EDGES_PER_WORKER

    def step(g, carry):
      base = base_w + g * CHUNK
      pltpu.sync_copy(src_hbm.at[pl.ds(base, CHUNK)], sidx)
      pltpu.sync_copy(dst_hbm.at[pl.ds(base, CHUNK)], didx)
      pltpu.async_copy(x_hbm.at[sidx], rows, sem).wait()
      pltpu.sync_copy(rows, acc.at[didx], add=True)
      if with_deg:
        pltpu.sync_copy(ones, degacc.at[didx], add=True)
      return carry
    lax.fori_loop(0, NUM_CHUNKS, step, 0)

    plsc.subcore_barrier()

    off = s * ROWS_PER_TILE
    pltpu.sync_copy(acc.at[pl.ds(off, ROWS_PER_TILE)],
                    part_hbm.at[c, pl.ds(off, ROWS_PER_TILE)])
    if with_deg:
      pltpu.sync_copy(degacc.at[pl.ds(off, ROWS_PER_TILE)],
                      deg_hbm.at[c, pl.ds(off, ROWS_PER_TILE)])

  return agg


def _make_tc_combine(relu, block_rows=1000):
  def body(x_ref, p_ref, dg_ref, ws_ref, wn_ref, b_ref, o_ref):
    agg = p_ref[0] + p_ref[1]
    deg16 = dg_ref[0] + dg_ref[1]
    inv = 1.0 / jnp.maximum(jnp.max(deg16, axis=1, keepdims=True), 1.0)
    h = jnp.dot(x_ref[...], ws_ref[...], preferred_element_type=jnp.float32)
    h = h + jnp.dot(agg * inv, wn_ref[...],
                    preferred_element_type=jnp.float32)
    h = h + b_ref[...]
    if relu:
      h = jnp.maximum(h, 0.0)
    o_ref[...] = h

  grid = (N_NODES // block_rows,)
  return pl.pallas_call(
      body,
      grid=grid,
      in_specs=[
          pl.BlockSpec((block_rows, D), lambda i: (i, 0)),
          pl.BlockSpec((NUM_CORES, block_rows, D), lambda i: (0, i, 0)),
          pl.BlockSpec((NUM_CORES, block_rows, LANES), lambda i: (0, i, 0)),
          pl.BlockSpec((D, D), lambda i: (0, 0)),
          pl.BlockSpec((D, D), lambda i: (0, 0)),
          pl.BlockSpec((1, D), lambda i: (0, 0)),
      ],
      out_specs=pl.BlockSpec((block_rows, D), lambda i: (i, 0)),
      out_shape=_SDS((N_NODES, D), jnp.float32),
  )


_agg_with_deg = _make_sc_agg(with_deg=True)
_agg_no_deg = _make_sc_agg(with_deg=False)
_combine_relu = _make_tc_combine(relu=True)
_combine_linear = _make_tc_combine(relu=False)


def kernel(in_feat, edge_index, W1_self, W1_neigh, b1, W2_self, W2_neigh, b2):
  src = edge_index[0].astype(jnp.int32)
  dst = edge_index[1].astype(jnp.int32)
  part1, degp = _agg_with_deg(in_feat, src, dst)
  h1 = _combine_relu(in_feat, part1, degp, W1_self, W1_neigh,
                     b1.reshape(1, D))
  part2 = _agg_no_deg(h1, src, dst)
  out = _combine_linear(h1, part2, degp, W2_self, W2_neigh,
                        b2.reshape(1, D))
  return out


# speedup vs baseline: 4.7868x; 4.7868x over previous
import functools

import jax
import jax.numpy as jnp
from jax import lax
from jax.experimental import pallas as pl
from jax.experimental.pallas import tpu as pltpu
from jax.experimental.pallas import tpu_sc as plsc

N_NODES = 10000
N_EDGES = 320000
D = 128
LANES = 16

NUM_CORES = 2
NUM_SUBCORES = 16
NUM_WORKERS = NUM_CORES * NUM_SUBCORES
EDGES_PER_WORKER = N_EDGES // NUM_WORKERS
CHUNK = 80
NUM_CHUNKS = EDGES_PER_WORKER // CHUNK
STRIPE = 624
TAIL = N_NODES - NUM_SUBCORES * STRIPE
ZROWS = 48

_SDS = jax.ShapeDtypeStruct


def _fill_zero(buf, nrows):
  zero16 = jnp.zeros((LANES,), jnp.float32)

  def body(k, carry):
    buf[k // (D // LANES), pl.ds((k % (D // LANES)) * LANES, LANES)] = zero16
    return carry
  lax.fori_loop(0, nrows * (D // LANES), body, 0)


def _zero_and_writeout_specs(s):
  return s * STRIPE


def _make_row_agg():
  mesh = plsc.VectorSubcoreMesh(
      core_axis_name="c", subcore_axis_name="s",
      num_cores=NUM_CORES, num_subcores=NUM_SUBCORES)

  @functools.partial(
      pl.kernel, mesh=mesh,
      out_type=_SDS((NUM_CORES, N_NODES, D), jnp.float32),
      scratch_types=(
          pltpu.VMEM_SHARED((N_NODES, D), jnp.float32),
          pltpu.VMEM((CHUNK,), jnp.int32),
          pltpu.VMEM((CHUNK,), jnp.int32),
          pltpu.VMEM((CHUNK, D), jnp.float32),
          pltpu.VMEM((ZROWS, D), jnp.float32),
          pltpu.SemaphoreType.DMA,
      ))
  def agg(x_hbm, src_hbm, dst_hbm, part_hbm, acc, sidx, didx, rows, zbuf, sem):
    c = lax.axis_index("c")
    s = lax.axis_index("s")
    wid = s * NUM_CORES + c

    _fill_zero(zbuf, ZROWS)
    off = s * STRIPE
    for r in range(STRIPE // ZROWS):
      pltpu.sync_copy(zbuf, acc.at[pl.ds(off + r * ZROWS, ZROWS)])

    @pl.when(s == NUM_SUBCORES - 1)
    def _zero_tail():
      pltpu.sync_copy(zbuf.at[pl.ds(0, TAIL)],
                      acc.at[pl.ds(NUM_SUBCORES * STRIPE, TAIL)])
    plsc.subcore_barrier()

    base_w = wid * EDGES_PER_WORKER

    def step(g, carry):
      base = base_w + g * CHUNK
      pltpu.sync_copy(src_hbm.at[pl.ds(base, CHUNK)], sidx)
      pltpu.sync_copy(dst_hbm.at[pl.ds(base, CHUNK)], didx)
      pltpu.async_copy(x_hbm.at[sidx], rows, sem).wait()
      pltpu.sync_copy(rows, acc.at[didx], add=True)
      return carry
    lax.fori_loop(0, NUM_CHUNKS, step, 0)

    plsc.subcore_barrier()
    pltpu.sync_copy(acc.at[pl.ds(off, STRIPE)],
                    part_hbm.at[c, pl.ds(off, STRIPE)])

    @pl.when(s == NUM_SUBCORES - 1)
    def _write_tail():
      toff = NUM_SUBCORES * STRIPE
      pltpu.sync_copy(acc.at[pl.ds(toff, TAIL)],
                      part_hbm.at[c, pl.ds(toff, TAIL)])

  return agg


def _make_deg():
  mesh = plsc.VectorSubcoreMesh(
      core_axis_name="c", subcore_axis_name="s",
      num_cores=NUM_CORES, num_subcores=NUM_SUBCORES)

  @functools.partial(
      pl.kernel, mesh=mesh,
      out_type=_SDS((NUM_CORES, N_NODES, D), jnp.float32),
      scratch_types=(
          pltpu.VMEM_SHARED((N_NODES, D), jnp.float32),
          pltpu.VMEM((CHUNK,), jnp.int32),
          pltpu.VMEM((CHUNK, D), jnp.float32),
          pltpu.VMEM((ZROWS, D), jnp.float32),
      ))
  def deg(dst_hbm, deg_hbm, acc, didx, ones, zbuf):
    c = lax.axis_index("c")
    s = lax.axis_index("s")
    wid = s * NUM_CORES + c

    _fill_zero(zbuf, ZROWS)
    one16 = jnp.full((LANES,), 1.0, jnp.float32)

    def ofill(k, carry):
      ones[k // (D // LANES), pl.ds((k % (D // LANES)) * LANES, LANES)] = one16
      return carry
    lax.fori_loop(0, CHUNK * (D // LANES), ofill, 0)

    off = s * STRIPE
    for r in range(STRIPE // ZROWS):
      pltpu.sync_copy(zbuf, acc.at[pl.ds(off + r * ZROWS, ZROWS)])

    @pl.when(s == NUM_SUBCORES - 1)
    def _zero_tail():
      pltpu.sync_copy(zbuf.at[pl.ds(0, TAIL)],
                      acc.at[pl.ds(NUM_SUBCORES * STRIPE, TAIL)])
    plsc.subcore_barrier()

    base_w = wid * EDGES_PER_WORKER

    def step(g, carry):
      base = base_w + g * CHUNK
      pltpu.sync_copy(dst_hbm.at[pl.ds(base, CHUNK)], didx)
      pltpu.sync_copy(ones, acc.at[didx], add=True)
      return carry
    lax.fori_loop(0, NUM_CHUNKS, step, 0)

    plsc.subcore_barrier()
    pltpu.sync_copy(acc.at[pl.ds(off, STRIPE)],
                    deg_hbm.at[c, pl.ds(off, STRIPE)])

    @pl.when(s == NUM_SUBCORES - 1)
    def _write_tail():
      toff = NUM_SUBCORES * STRIPE
      pltpu.sync_copy(acc.at[pl.ds(toff, TAIL)],
                      deg_hbm.at[c, pl.ds(toff, TAIL)])

  return deg


def _make_tc_combine(relu, block_rows=1000):
  def body(x_ref, p_ref, dg_ref, ws_ref, wn_ref, b_ref, o_ref):
    agg = p_ref[0] + p_ref[1]
    deg = dg_ref[0] + dg_ref[1]
    inv = 1.0 / jnp.maximum(jnp.max(deg, axis=1, keepdims=True), 1.0)
    h = jnp.dot(x_ref[...], ws_ref[...], preferred_element_type=jnp.float32)
    h = h + jnp.dot(agg * inv, wn_ref[...],
                    preferred_element_type=jnp.float32)
    h = h + b_ref[...]
    if relu:
      h = jnp.maximum(h, 0.0)
    o_ref[...] = h

  return pl.pallas_call(
      body,
      grid=(N_NODES // block_rows,),
      in_specs=[
          pl.BlockSpec((block_rows, D), lambda i: (i, 0)),
          pl.BlockSpec((NUM_CORES, block_rows, D), lambda i: (0, i, 0)),
          pl.BlockSpec((NUM_CORES, block_rows, D), lambda i: (0, i, 0)),
          pl.BlockSpec((D, D), lambda i: (0, 0)),
          pl.BlockSpec((D, D), lambda i: (0, 0)),
          pl.BlockSpec((1, D), lambda i: (0, 0)),
      ],
      out_specs=pl.BlockSpec((block_rows, D), lambda i: (i, 0)),
      out_shape=_SDS((N_NODES, D), jnp.float32),
  )


_get_row_agg = functools.lru_cache(maxsize=None)(_make_row_agg)
_get_deg = functools.lru_cache(maxsize=None)(_make_deg)
_combine_relu = _make_tc_combine(relu=True)
_combine_linear = _make_tc_combine(relu=False)


def kernel(in_feat, edge_index, W1_self, W1_neigh, b1, W2_self, W2_neigh, b2):
  src = edge_index[0].astype(jnp.int32)
  dst = edge_index[1].astype(jnp.int32)
  degp = _get_deg()(dst)
  part1 = _get_row_agg()(in_feat, src, dst)
  h1 = _combine_relu(in_feat, part1, degp, W1_self, W1_neigh,
                     b1.reshape(1, D))
  part2 = _get_row_agg()(h1, src, dst)
  out = _combine_linear(h1, part2, degp, W2_self, W2_neigh,
                        b2.reshape(1, D))
  return out


# speedup vs baseline: 7.7160x; 1.6119x over previous
import functools

import jax
import jax.numpy as jnp
from jax import lax
from jax.experimental import pallas as pl
from jax.experimental.pallas import tpu as pltpu
from jax.experimental.pallas import tpu_sc as plsc

N_NODES = 10000
N_EDGES = 320000
D = 128
LANES = 16

NUM_CORES = 2
NUM_SUBCORES = 16
NUM_WORKERS = NUM_CORES * NUM_SUBCORES
EDGES_PER_WORKER = N_EDGES // NUM_WORKERS
CHUNK = 80
NUM_CHUNKS = EDGES_PER_WORKER // CHUNK
STRIPE = 624
TAIL = N_NODES - NUM_SUBCORES * STRIPE
ZROWS = 48

_SDS = jax.ShapeDtypeStruct


def _fill_zero(buf, nrows):
  zero16 = jnp.zeros((LANES,), jnp.float32)

  def body(k, carry):
    buf[k // (D // LANES), pl.ds((k % (D // LANES)) * LANES, LANES)] = zero16
    return carry
  lax.fori_loop(0, nrows * (D // LANES), body, 0)


def _zero_and_writeout_specs(s):
  return s * STRIPE


def _make_row_agg():
  mesh = plsc.VectorSubcoreMesh(
      core_axis_name="c", subcore_axis_name="s",
      num_cores=NUM_CORES, num_subcores=NUM_SUBCORES)

  @functools.partial(
      pl.kernel, mesh=mesh,
      out_type=_SDS((NUM_CORES, N_NODES, D), jnp.float32),
      scratch_types=(
          pltpu.VMEM_SHARED((N_NODES, D), jnp.float32),
          pltpu.VMEM((CHUNK,), jnp.int32),
          pltpu.VMEM((CHUNK,), jnp.int32),
          pltpu.VMEM((CHUNK,), jnp.int32),
          pltpu.VMEM((CHUNK,), jnp.int32),
          pltpu.VMEM((CHUNK, D), jnp.float32),
          pltpu.VMEM((CHUNK, D), jnp.float32),
          pltpu.VMEM((ZROWS, D), jnp.float32),
          pltpu.SemaphoreType.DMA,
          pltpu.SemaphoreType.DMA,
      ))
  def agg(x_hbm, src_hbm, dst_hbm, part_hbm, acc,
          sidx0, sidx1, didx0, didx1, rows0, rows1, zbuf, sem0, sem1):
    c = lax.axis_index("c")
    s = lax.axis_index("s")
    wid = s * NUM_CORES + c

    _fill_zero(zbuf, ZROWS)
    off = s * STRIPE
    for r in range(STRIPE // ZROWS):
      pltpu.sync_copy(zbuf, acc.at[pl.ds(off + r * ZROWS, ZROWS)])

    @pl.when(s == NUM_SUBCORES - 1)
    def _zero_tail():
      pltpu.sync_copy(zbuf.at[pl.ds(0, TAIL)],
                      acc.at[pl.ds(NUM_SUBCORES * STRIPE, TAIL)])
    plsc.subcore_barrier()

    base_w = wid * EDGES_PER_WORKER

    def load_idx(sb, db, base):
      pltpu.sync_copy(src_hbm.at[pl.ds(base, CHUNK)], sb)
      pltpu.sync_copy(dst_hbm.at[pl.ds(base, CHUNK)], db)

    load_idx(sidx0, didx0, base_w)
    pltpu.async_copy(x_hbm.at[sidx0], rows0, sem0)

    def pair(k, carry):
      load_idx(sidx1, didx1, base_w + (2 * k + 1) * CHUNK)
      pltpu.async_copy(x_hbm.at[sidx1], rows1, sem1)
      pltpu.make_async_copy(x_hbm.at[sidx0], rows0, sem0).wait()
      pltpu.sync_copy(rows0, acc.at[didx0], add=True)
      load_idx(sidx0, didx0, base_w + (2 * k + 2) * CHUNK)
      pltpu.async_copy(x_hbm.at[sidx0], rows0, sem0)
      pltpu.make_async_copy(x_hbm.at[sidx1], rows1, sem1).wait()
      pltpu.sync_copy(rows1, acc.at[didx1], add=True)
      return carry
    lax.fori_loop(0, (NUM_CHUNKS - 1) // 2, pair, 0)
    pltpu.make_async_copy(x_hbm.at[sidx0], rows0, sem0).wait()
    pltpu.sync_copy(rows0, acc.at[didx0], add=True)

    plsc.subcore_barrier()
    pltpu.sync_copy(acc.at[pl.ds(off, STRIPE)],
                    part_hbm.at[c, pl.ds(off, STRIPE)])

    @pl.when(s == NUM_SUBCORES - 1)
    def _write_tail():
      toff = NUM_SUBCORES * STRIPE
      pltpu.sync_copy(acc.at[pl.ds(toff, TAIL)],
                      part_hbm.at[c, pl.ds(toff, TAIL)])

  return agg


def _make_deg():
  mesh = plsc.VectorSubcoreMesh(
      core_axis_name="c", subcore_axis_name="s",
      num_cores=NUM_CORES, num_subcores=NUM_SUBCORES)

  @functools.partial(
      pl.kernel, mesh=mesh,
      out_type=_SDS((NUM_CORES, N_NODES, D), jnp.float32),
      scratch_types=(
          pltpu.VMEM_SHARED((N_NODES, D), jnp.float32),
          pltpu.VMEM((CHUNK,), jnp.int32),
          pltpu.VMEM((CHUNK,), jnp.int32),
          pltpu.VMEM((CHUNK, D), jnp.float32),
          pltpu.VMEM((ZROWS, D), jnp.float32),
          pltpu.SemaphoreType.DMA,
          pltpu.SemaphoreType.DMA,
      ))
  def deg(dst_hbm, deg_hbm, acc, didx0, didx1, ones, zbuf, sem0, sem1):
    c = lax.axis_index("c")
    s = lax.axis_index("s")
    wid = s * NUM_CORES + c

    _fill_zero(zbuf, ZROWS)
    one16 = jnp.full((LANES,), 1.0, jnp.float32)

    def ofill(k, carry):
      ones[k // (D // LANES), pl.ds((k % (D // LANES)) * LANES, LANES)] = one16
      return carry
    lax.fori_loop(0, CHUNK * (D // LANES), ofill, 0)

    off = s * STRIPE
    for r in range(STRIPE // ZROWS):
      pltpu.sync_copy(zbuf, acc.at[pl.ds(off + r * ZROWS, ZROWS)])

    @pl.when(s == NUM_SUBCORES - 1)
    def _zero_tail():
      pltpu.sync_copy(zbuf.at[pl.ds(0, TAIL)],
                      acc.at[pl.ds(NUM_SUBCORES * STRIPE, TAIL)])
    plsc.subcore_barrier()

    base_w = wid * EDGES_PER_WORKER

    pltpu.sync_copy(dst_hbm.at[pl.ds(base_w, CHUNK)], didx0)
    pltpu.async_copy(ones, acc.at[didx0], sem0, add=True)

    def pair(k, carry):
      pltpu.sync_copy(dst_hbm.at[pl.ds(base_w + (2 * k + 1) * CHUNK, CHUNK)],
                      didx1)
      pltpu.async_copy(ones, acc.at[didx1], sem1, add=True)
      pltpu.make_async_copy(ones, acc.at[didx0], sem0).wait()
      pltpu.sync_copy(dst_hbm.at[pl.ds(base_w + (2 * k + 2) * CHUNK, CHUNK)],
                      didx0)
      pltpu.async_copy(ones, acc.at[didx0], sem0, add=True)
      pltpu.make_async_copy(ones, acc.at[didx1], sem1).wait()
      return carry
    lax.fori_loop(0, (NUM_CHUNKS - 1) // 2, pair, 0)
    pltpu.make_async_copy(ones, acc.at[didx0], sem0).wait()

    plsc.subcore_barrier()
    pltpu.sync_copy(acc.at[pl.ds(off, STRIPE)],
                    deg_hbm.at[c, pl.ds(off, STRIPE)])

    @pl.when(s == NUM_SUBCORES - 1)
    def _write_tail():
      toff = NUM_SUBCORES * STRIPE
      pltpu.sync_copy(acc.at[pl.ds(toff, TAIL)],
                      deg_hbm.at[c, pl.ds(toff, TAIL)])

  return deg


def _make_tc_combine(relu, block_rows=1000):
  def body(x_ref, p_ref, dg_ref, ws_ref, wn_ref, b_ref, o_ref):
    agg = p_ref[0] + p_ref[1]
    deg = dg_ref[0] + dg_ref[1]
    inv = 1.0 / jnp.maximum(jnp.max(deg, axis=1, keepdims=True), 1.0)
    h = jnp.dot(x_ref[...], ws_ref[...], preferred_element_type=jnp.float32)
    h = h + jnp.dot(agg * inv, wn_ref[...],
                    preferred_element_type=jnp.float32)
    h = h + b_ref[...]
    if relu:
      h = jnp.maximum(h, 0.0)
    o_ref[...] = h

  return pl.pallas_call(
      body,
      grid=(N_NODES // block_rows,),
      in_specs=[
          pl.BlockSpec((block_rows, D), lambda i: (i, 0)),
          pl.BlockSpec((NUM_CORES, block_rows, D), lambda i: (0, i, 0)),
          pl.BlockSpec((NUM_CORES, block_rows, D), lambda i: (0, i, 0)),
          pl.BlockSpec((D, D), lambda i: (0, 0)),
          pl.BlockSpec((D, D), lambda i: (0, 0)),
          pl.BlockSpec((1, D), lambda i: (0, 0)),
      ],
      out_specs=pl.BlockSpec((block_rows, D), lambda i: (i, 0)),
      out_shape=_SDS((N_NODES, D), jnp.float32),
  )


_get_row_agg = functools.lru_cache(maxsize=None)(_make_row_agg)
_get_deg = functools.lru_cache(maxsize=None)(_make_deg)
_combine_relu = _make_tc_combine(relu=True)
_combine_linear = _make_tc_combine(relu=False)


def kernel(in_feat, edge_index, W1_self, W1_neigh, b1, W2_self, W2_neigh, b2):
  src = edge_index[0].astype(jnp.int32)
  dst = edge_index[1].astype(jnp.int32)
  degp = _get_deg()(dst)
  part1 = _get_row_agg()(in_feat, src, dst)
  h1 = _combine_relu(in_feat, part1, degp, W1_self, W1_neigh,
                     b1.reshape(1, D))
  part2 = _get_row_agg()(h1, src, dst)
  out = _combine_linear(h1, part2, degp, W2_self, W2_neigh,
                        b2.reshape(1, D))
  return out
